# baseline (speedup 1.0000x reference)
import jax
import jax.numpy as jnp
from jax.experimental import pallas as pl


def _spec_kernel(t_ref, f_ref, e_ref, x_ref, o_ref):
    t = t_ref[0]
    f = f_ref[0]
    e = e_ref[...]
    o_ref[0] = jnp.where(f, jnp.float32(0.0), jnp.where(t, e, x_ref[0]))


def kernel(x, masked_spec_embed, mask_time, flip_mask, mask_feature):
    B, L, D = x.shape
    BL = 512
    NL = L // BL
    t = (mask_time & flip_mask).reshape(B * NL, BL, 1)
    f = mask_feature.reshape(B, 1, D)
    e = masked_spec_embed.reshape(1, D).astype(x.dtype)

    grid = (B, NL)
    return pl.pallas_call(
        _spec_kernel,
        grid=grid,
        in_specs=[
            pl.BlockSpec((1, BL, 1), lambda b, l: (b * NL + l, 0, 0)),
            pl.BlockSpec((1, 1, D), lambda b, l: (b, 0, 0)),
            pl.BlockSpec((1, D), lambda b, l: (0, 0)),
            pl.BlockSpec((1, BL, D), lambda b, l: (b, l, 0)),
        ],
        out_specs=pl.BlockSpec((1, BL, D), lambda b, l: (b, l, 0)),
        out_shape=jax.ShapeDtypeStruct((B, L, D), x.dtype),
    )(t, f, e, x)


# speedup vs baseline: 1.1826x; 1.1826x over previous
import jax
import jax.numpy as jnp
from jax.experimental import pallas as pl

_SB = 2


def _spec_kernel(t_ref, f_ref, e_ref, x_ref, o_ref):
    e = e_ref[...]
    for i in range(_SB):
        t = t_ref[i]
        f = f_ref[i]
        o_ref[i] = jnp.where(f, jnp.float32(0.0), jnp.where(t, e, x_ref[i]))


def kernel(x, masked_spec_embed, mask_time, flip_mask, mask_feature):
    B, L, D = x.shape
    t = (mask_time & flip_mask).reshape(B, L, 1)
    f = mask_feature.reshape(B, 1, D)
    e = masked_spec_embed.reshape(1, D).astype(x.dtype)

    grid = (B // _SB,)
    return pl.pallas_call(
        _spec_kernel,
        grid=grid,
        in_specs=[
            pl.BlockSpec((_SB, L, 1), lambda b: (b, 0, 0)),
            pl.BlockSpec((_SB, 1, D), lambda b: (b, 0, 0)),
            pl.BlockSpec((1, D), lambda b: (0, 0)),
            pl.BlockSpec((_SB, L, D), lambda b: (b, 0, 0)),
        ],
        out_specs=pl.BlockSpec((_SB, L, D), lambda b: (b, 0, 0)),
        out_shape=jax.ShapeDtypeStruct((B, L, D), x.dtype),
    )(t, f, e, x)
